# baseline (speedup 1.0000x reference)
import functools

import numpy as np
import jax
import jax.numpy as jnp
from jax import lax
from jax.experimental import pallas as pl
from jax.experimental.pallas import tpu as pltpu
from jax.experimental.pallas import tpu_sc as plsc

_DIM = 384
_HEADS = 8
_DH = 64
_INNER = 512
_TOPK = 512
_NQ = 1024
_NKV = 4096
_DHP = 128
_HIGH = jax.lax.Precision.HIGHEST


_SAMPLE_IDX = np.array([
    [6,8,11,13,17,18,20,23,25,26,28,34,37,41,45,46,48,51,56,58,60,62,66,67,69,70,71,74,80,83,84,87,91,92,93,95,97,99,101,108,112,114,120,125,127,128,131,134,135,136,143,145,146,147,149,153,154,164,168,171,172,173,183,184,185,186,192,193,196,198,199,201,206,207,209,212,219,223,224,225,226,228,232,235,238,239,240,247,248,249,250,252,254,258,261,264,266,267,268,269,276,277,278,279,284,285,288,291,293,295,297,299,300,302,303,304,308,309,311,316,318,321,324,325,332,334,335,337,338,342,345,348,352,353,354,356,357,361,363,365,370,371,372,377,381,383,385,386,388,389,395,397,402,406,408,409,410,411,414,415,420,422,424,428,431,433,436,437,438,439,440,441,446,448,451,453,456,459,460,463,465,467,469,472,474,475,479,484,487,489,492,495,496,497,500,501,502,503,505,508,518,519,523,525,527,528,529,530,535,539,543,544,545,546,548,549,552,556,561,562,564,566,567,568,571,574,576,578,579,580,587,588,591,592,593,595,596,598,601,604,606,608,609,611,612,617,619,620,625,626,629,634,641,643,647,648,650,651,653,654,660,661,663,665,667,674,676,682,683,684,686,691,692,694,695,696,698,700,706,707,714,716,720,729,730,731,733,735,737,740,742,749,750,756,758,759,760,762,764,767,771,772,774,775,776,779,784,786,788,789,790,792,793,795,798,799,800,802,803,804,807,808,812,814,815,816,817,819,820,823,824,827,828,830,832,833,834,835,837,839,840,842,845,849,856,857,859,860,861,862,865,869,871,874,878,889,891,892,894,895,898,902,904,906,913,922,924,928,930,933,938,944,946,948,953,955,961,967,969,971,972,974,976,978,980,985,986,993,999,1000,1003,1005,1008,1010,1011,1013,1015,1016,1018,1019,1021,1022,1023,6,6,6,6,6,6,6,6,6,6,6,6,6],
    [2,5,11,12,14,37,40,41,42,44,46,48,49,53,55,56,59,64,65,68,70,75,77,81,82,86,91,96,97,98,101,104,107,111,114,117,124,125,128,129,130,135,136,138,140,147,150,152,153,154,156,160,161,163,164,167,169,172,175,183,184,186,187,188,189,191,196,197,201,206,208,210,212,213,214,216,218,220,221,223,227,229,230,236,237,240,246,248,249,251,254,256,258,260,264,266,267,270,272,277,278,279,280,281,286,287,290,294,296,298,299,302,304,307,309,317,318,320,325,326,328,330,331,332,335,336,338,339,340,343,345,347,348,355,357,358,364,365,366,367,369,370,371,374,376,378,381,383,387,389,391,399,400,405,407,408,412,414,415,419,421,426,429,430,431,433,436,437,444,445,449,450,453,454,457,458,459,462,463,468,469,475,476,477,478,480,481,485,487,490,491,494,496,498,501,507,510,511,512,515,520,525,528,530,533,534,541,543,544,547,548,554,556,557,558,559,561,568,569,570,574,583,586,592,593,597,599,601,605,606,609,610,611,613,618,620,623,624,625,633,635,637,638,640,641,642,643,645,649,653,654,657,660,661,663,664,669,671,673,674,679,689,698,700,703,704,709,710,711,712,714,718,719,721,722,723,726,727,729,730,732,733,738,740,741,742,745,746,750,754,756,757,758,762,768,770,773,774,775,778,781,784,785,787,789,791,792,793,794,799,802,804,810,813,814,823,824,825,828,831,838,842,843,844,847,849,850,853,858,859,860,862,863,864,865,873,874,875,876,880,883,885,886,887,891,892,894,895,897,899,903,905,907,910,915,923,926,930,934,937,938,940,943,947,949,959,960,962,963,965,966,968,972,974,984,987,988,989,990,991,1001,1003,1005,1006,1008,1010,1011,1012,1016,1017,1018,1019,2,2,2,2,2,2,2,2,2,2,2,2,2,2,2,2,2,2,2,2,2,2,2,2],
    [2,4,6,8,9,10,15,18,21,23,24,25,26,32,34,37,39,43,44,46,47,48,50,58,59,61,63,64,67,71,72,77,82,84,85,89,93,94,95,97,98,101,103,106,112,113,120,122,124,127,129,131,132,133,134,135,137,138,151,155,156,159,162,163,165,166,171,172,173,178,182,186,188,189,197,198,203,206,208,211,215,217,220,221,224,226,229,233,237,245,250,254,259,260,264,265,267,268,270,271,272,274,276,279,280,281,284,290,293,294,296,297,299,302,304,305,308,312,317,322,324,325,326,327,330,333,336,340,342,345,346,347,349,352,353,356,357,358,360,367,368,369,372,378,379,382,383,385,386,387,390,392,394,395,396,397,398,399,402,403,404,405,407,408,410,411,412,414,417,418,419,421,422,428,429,434,435,437,438,444,452,455,457,458,461,462,465,469,470,471,475,479,480,481,483,486,487,488,492,498,501,503,505,511,520,521,524,527,541,542,544,548,550,551,555,556,557,560,564,567,569,573,575,579,580,581,584,585,587,589,590,591,593,595,596,597,599,607,613,617,621,623,624,625,626,631,638,639,640,642,646,647,649,650,652,656,657,660,661,667,669,670,671,673,680,681,683,684,686,690,691,692,693,694,695,696,697,701,707,708,709,710,713,714,715,717,719,720,721,722,729,731,734,735,736,743,744,750,751,754,757,759,760,762,763,764,765,766,768,774,775,777,782,785,786,787,788,789,792,794,796,797,799,802,803,811,812,813,815,818,820,821,823,824,825,826,828,829,831,833,836,837,838,839,840,841,845,846,848,850,852,853,854,856,859,860,864,867,869,870,874,877,878,879,882,883,884,890,891,892,893,903,905,908,911,914,919,920,925,926,927,928,930,931,936,938,939,940,946,948,949,950,953,956,959,962,964,969,970,973,974,985,988,990,993,994,996,997,999,1002,1004,1007,1010,1016,2,2],
    [0,2,4,10,13,17,18,19,20,28,30,33,34,37,38,42,43,44,45,48,51,53,55,56,57,61,62,64,65,68,79,81,82,83,87,92,98,99,103,104,106,111,112,114,117,124,126,127,133,134,135,141,142,144,148,149,151,155,156,157,158,161,162,163,166,167,171,173,178,179,184,186,188,189,190,191,199,206,209,210,213,217,221,224,225,226,229,230,233,235,236,237,238,240,245,249,252,256,261,262,263,264,265,267,270,273,274,278,280,281,282,285,288,289,291,293,297,299,301,302,303,308,310,311,313,314,321,322,326,327,329,330,335,336,344,347,349,352,358,363,367,369,380,383,386,387,389,397,402,404,406,407,408,410,411,413,414,415,423,424,427,429,430,436,440,443,444,446,447,451,453,456,459,463,465,469,473,474,475,476,480,481,482,483,485,486,488,489,491,495,497,498,500,501,503,505,506,507,508,509,510,519,522,523,526,527,529,530,531,532,533,534,536,537,538,541,543,544,546,552,553,554,557,559,560,561,563,566,569,571,572,575,576,583,586,587,588,592,593,595,596,598,601,602,603,604,606,610,611,613,614,617,621,622,623,624,627,628,637,639,642,643,645,648,652,656,662,663,671,674,675,680,689,691,694,695,699,700,701,703,705,707,710,712,713,716,718,719,721,722,723,726,728,732,733,734,735,744,745,746,749,751,753,760,767,768,770,772,775,780,783,785,786,789,799,800,801,804,806,814,818,822,823,827,828,830,832,837,838,839,841,842,843,849,856,859,861,862,863,865,870,876,880,888,889,891,895,896,897,899,900,906,907,908,910,911,912,915,916,918,919,921,922,929,930,933,934,936,938,939,941,943,944,946,950,952,955,958,960,963,964,966,969,971,973,974,975,976,977,978,979,980,981,985,986,988,990,994,998,999,1000,1001,1002,1008,1010,1014,1015,1017,1021,1022,0,0,0,0,0,0],
    [0,1,2,3,5,7,8,12,18,22,23,24,25,26,30,33,41,43,44,47,50,56,57,59,60,61,62,65,66,67,69,71,73,74,78,79,80,81,82,87,90,92,94,96,98,100,102,103,106,107,108,113,117,121,122,125,126,127,128,130,133,136,143,145,148,149,150,152,154,156,161,165,167,168,170,172,179,181,184,185,186,188,196,204,207,215,217,218,229,230,233,234,235,236,237,242,244,245,246,247,252,254,256,258,261,262,264,266,267,269,272,273,274,275,277,285,287,295,296,297,299,300,302,303,306,310,311,312,313,314,315,319,321,330,334,335,337,338,340,355,358,359,361,362,365,367,368,369,371,374,375,376,380,382,383,384,389,393,395,400,401,403,405,407,408,410,413,415,417,419,423,424,425,426,428,430,432,435,438,440,442,443,450,452,455,459,462,464,465,471,472,474,475,476,477,478,480,483,488,489,492,493,495,496,498,499,500,503,505,512,513,514,518,520,524,531,532,538,539,541,544,551,552,556,557,558,560,563,564,567,569,582,586,588,591,592,596,602,605,606,607,614,615,621,625,627,631,633,638,639,643,644,646,647,650,651,654,656,659,662,665,672,674,678,682,686,687,689,692,696,697,698,703,705,707,711,715,718,721,723,726,728,732,736,739,741,743,758,763,765,770,772,774,778,785,786,787,790,794,799,800,806,812,813,816,823,824,825,829,832,833,841,844,846,847,851,860,861,862,863,864,865,866,869,870,874,876,882,884,885,886,887,891,894,895,896,898,900,901,903,904,906,907,908,912,913,914,916,917,920,924,926,928,930,931,932,934,937,939,940,941,942,946,950,951,953,956,958,960,962,963,964,966,969,973,977,978,979,980,981,984,988,994,1000,1001,1005,1007,1010,1011,1013,1016,1020,1021,1023,0,0,0,0,0,0,0,0,0,0,0,0,0,0,0,0,0,0,0,0,0,0],
    [8,9,11,14,15,17,18,19,20,22,35,36,38,39,40,42,43,45,49,50,55,56,61,62,63,68,71,75,78,79,84,87,97,99,100,102,103,105,106,114,124,127,130,132,136,137,138,140,144,145,146,148,151,155,156,157,162,171,174,178,180,183,184,186,187,190,196,197,198,204,205,209,210,211,213,214,221,222,225,228,229,233,235,239,241,242,243,244,247,248,250,252,253,254,255,256,257,260,263,266,268,270,271,273,274,276,277,278,281,284,285,286,288,289,291,297,298,302,307,309,312,313,317,318,320,323,324,327,328,332,335,346,347,359,364,365,369,371,374,375,377,380,382,384,386,387,388,389,391,392,395,400,404,405,406,409,413,415,417,418,420,422,424,425,426,428,430,431,432,434,437,438,441,442,445,446,451,452,453,455,456,464,466,468,470,471,474,475,476,477,478,479,482,483,492,495,497,498,501,502,503,510,511,515,516,519,523,524,525,527,530,531,532,535,538,539,541,542,543,546,557,559,565,567,570,573,578,581,584,587,592,596,597,598,601,602,604,605,607,611,614,618,620,626,627,628,630,631,638,641,643,644,647,651,652,658,659,662,665,668,673,674,676,677,678,679,682,684,685,694,700,701,704,706,711,712,714,715,717,721,724,725,726,732,734,735,736,738,742,743,744,746,749,750,752,757,761,763,765,766,771,773,775,777,779,780,781,787,788,789,794,795,798,801,803,805,809,812,815,816,821,822,825,826,827,828,833,834,836,838,840,842,844,847,852,854,855,857,858,868,869,870,871,873,874,875,877,880,881,883,885,888,890,896,897,898,899,903,904,910,912,914,916,917,919,923,924,926,927,933,936,938,943,944,950,959,960,962,963,968,974,975,977,983,984,985,986,987,989,990,992,994,998,999,1001,1002,1003,1006,1007,1010,1011,1013,1022,8,8,8,8,8,8,8,8,8,8,8,8,8],
    [0,1,3,4,5,8,9,11,12,15,16,18,26,30,32,33,42,44,48,53,54,56,58,59,60,63,68,70,72,78,80,82,83,85,90,95,96,97,101,103,105,109,111,121,122,134,136,139,140,141,144,150,152,157,158,159,160,163,164,169,171,172,173,174,175,178,181,182,183,184,186,189,192,193,195,197,198,199,200,206,209,211,213,216,218,219,220,222,225,227,231,233,234,235,236,238,241,245,247,248,249,250,255,256,258,261,262,265,269,270,276,277,284,286,292,293,295,296,300,306,307,308,309,311,313,320,328,332,333,334,339,340,342,345,346,349,352,355,357,359,360,362,368,370,375,376,378,382,385,386,387,391,397,407,411,412,416,417,425,430,431,436,438,439,440,441,443,444,450,453,454,456,457,459,461,462,464,466,467,469,470,471,473,476,478,479,480,481,482,483,490,491,493,494,497,498,499,500,501,505,508,509,511,514,515,519,522,524,529,530,534,537,550,551,555,557,558,559,560,561,568,570,573,580,589,593,595,597,598,599,602,603,604,610,623,624,625,627,629,634,636,640,641,645,647,649,656,657,658,659,660,661,662,666,667,668,675,677,680,683,684,685,686,689,693,697,700,703,711,715,716,721,724,725,728,729,732,737,738,739,740,745,746,747,748,750,752,754,755,756,757,759,761,764,772,779,780,784,786,791,792,794,795,802,805,806,807,809,810,812,813,815,818,819,822,824,825,833,834,836,838,839,840,842,843,846,847,848,849,852,857,859,860,868,871,872,877,879,880,881,882,883,885,886,893,895,898,900,901,903,906,907,911,913,914,916,917,925,926,928,929,932,933,938,940,947,950,954,955,958,959,963,964,967,968,970,974,975,976,977,979,981,984,987,991,992,993,994,997,998,999,1000,1001,1003,1005,1008,1009,1010,1014,1017,1018,0,0,0,0,0,0,0,0,0,0,0,0,0,0,0],
    [0,6,14,17,20,21,26,27,28,30,35,38,40,41,42,45,52,57,59,61,62,63,66,71,73,76,77,79,80,83,86,87,88,89,90,91,96,100,103,105,108,110,117,118,122,123,127,130,131,132,134,135,136,139,140,143,146,149,150,154,158,159,160,175,177,181,182,189,191,193,198,199,202,204,206,207,209,210,212,213,216,217,219,221,225,227,230,234,240,243,244,246,247,250,254,264,265,266,268,269,270,275,278,281,283,284,287,288,289,293,294,295,301,302,304,305,309,312,320,322,327,331,332,336,338,343,347,350,355,359,360,361,363,370,371,372,373,376,377,379,385,391,392,394,395,397,399,401,406,417,421,425,429,432,438,439,440,441,443,445,446,449,456,457,458,462,470,472,476,477,484,486,488,489,494,499,501,504,505,508,522,526,530,533,536,537,538,539,540,544,545,549,550,552,553,554,557,562,564,567,568,569,573,574,578,579,580,581,583,585,589,590,591,593,594,595,597,599,600,602,605,607,608,612,614,616,617,625,626,627,636,638,639,640,643,652,653,655,656,657,660,661,664,667,670,678,680,681,684,687,689,691,693,694,695,696,701,703,704,711,712,715,718,720,721,722,723,725,727,729,730,731,732,734,736,739,740,743,747,749,750,751,752,755,757,759,760,761,763,765,767,770,771,773,774,776,777,789,790,794,795,797,799,800,801,805,806,807,810,812,814,817,818,819,822,825,829,830,835,842,844,850,854,858,859,860,861,862,863,865,867,868,870,873,875,876,879,880,881,883,885,893,894,896,897,898,899,901,906,909,910,911,915,917,918,921,923,924,929,930,933,935,938,946,947,948,950,952,956,957,958,961,965,968,969,971,973,977,979,982,985,990,993,1000,1001,1002,1004,1011,1012,1013,1016,1017,1020,1021,1022,0,0,0,0,0,0,0,0,0,0,0,0,0,0,0,0,0,0,0,0,0],
], dtype=np.int32)

_NS = _SAMPLE_IDX.shape[1]
_FLAT_IDX = (_SAMPLE_IDX + (np.arange(_HEADS, dtype=np.int32) * _NQ)[:, None]
             ).reshape(-1).astype(np.int32)
_NSC = 32
_PER_W = _FLAT_IDX.shape[0] // _NSC


def _chan_ln(x, g, b):
    m = jnp.mean(x, axis=0, keepdims=True)
    d = x - m
    var = jnp.mean(d * d, axis=0, keepdims=True)
    return d / jnp.sqrt(var + 1e-5) * g + b


def _prep_body(ctx_ref, qs_ref, wkv_ref, wq_ref, cg_ref, cb_ref, qg_ref,
               qb_ref, k_ref, v_ref, qhm_ref):
    ctxn = _chan_ln(ctx_ref[...], cg_ref[...], cb_ref[...])
    qsn = _chan_ln(qs_ref[...], qg_ref[...], qb_ref[...])

    kv = lax.dot_general(wkv_ref[...], ctxn, (((1,), (0,)), ((), ())),
                         preferred_element_type=jnp.float32)
    q = lax.dot_general(wq_ref[...], qsn, (((1,), (0,)), ((), ())),
                        preferred_element_type=jnp.float32)

    def l2n(x, n_pos):
        xr = x.reshape(_HEADS, _DH, n_pos)
        n = jnp.sqrt(jnp.sum(xr * xr, axis=1, keepdims=True))
        return (xr / jnp.maximum(n, 1e-12)).reshape(_HEADS * _DH, n_pos)

    k_ref[...] = l2n(kv[:_INNER], _NKV)
    v_ref[...] = kv[_INNER:]
    qn = l2n(q, _NQ)

    rows = lax.broadcasted_iota(jnp.int32, (_DH, _DHP), 0)
    cols = lax.broadcasted_iota(jnp.int32, (_DH, _DHP), 1)
    eye = (rows == cols).astype(jnp.float32)
    for h in range(_HEADS):
        qh = qn[h * _DH:(h + 1) * _DH]
        qht = lax.dot_general(qh, eye, (((0,), (0,)), ((), ())),
                              preferred_element_type=jnp.float32,
                              precision=_HIGH)
        qhm_ref[h * _NQ:(h + 1) * _NQ, :] = qht


def _prep_call(ctx2, qs2, W_kv, W_q, cg, cb, qg, qb, interpret=False):
    return pl.pallas_call(
        _prep_body,
        out_shape=[
            jax.ShapeDtypeStruct((_INNER, _NKV), jnp.float32),
            jax.ShapeDtypeStruct((_INNER, _NKV), jnp.float32),
            jax.ShapeDtypeStruct((_HEADS * _NQ, _DHP), jnp.float32),
        ],
        interpret=interpret,
    )(ctx2, qs2, W_kv, W_q, cg, cb, qg, qb)


@functools.cache
def _sc_gather_fn():
    @functools.partial(
        pl.kernel,
        out_type=jax.ShapeDtypeStruct((_HEADS * _NS, _DHP), jnp.float32),
        mesh=plsc.VectorSubcoreMesh(core_axis_name="c", subcore_axis_name="s"),
        scratch_types=[
            pltpu.VMEM((_PER_W,), jnp.int32),
            pltpu.VMEM((_PER_W, _DHP), jnp.float32),
            pltpu.SemaphoreType.DMA,
        ],
    )
    def _sc_gather(qhm_hbm, idx_hbm, out_hbm, idx_v, rows_v, sem):
        wid = lax.axis_index("s") * 2 + lax.axis_index("c")
        base = wid * _PER_W
        pltpu.sync_copy(idx_hbm.at[pl.ds(base, _PER_W)], idx_v)
        pltpu.async_copy(qhm_hbm.at[idx_v], rows_v, sem).wait()
        pltpu.sync_copy(rows_v, out_hbm.at[pl.ds(base, _PER_W)])

    return _sc_gather


_NQB = 4
_QB = _NQ // _NQB
_SCW = 16


def _main_body(k_ref, v_ref, qh_ref, qsm_ref, wo_ref, qs_ref, og_ref, ob_ref,
               gam_ref, out_ref, acc_ref, keep_ref, qsmt_ref):
    h = pl.program_id(0)
    qb = pl.program_id(1)
    kh = k_ref[...]

    @pl.when(qb == 0)
    def _():
        rows = lax.broadcasted_iota(jnp.int32, (_DH, _DHP), 0)
        cols = lax.broadcasted_iota(jnp.int32, (_DH, _DHP), 1)
        sel = (rows == cols).astype(jnp.float32)
        qsm_t = lax.dot_general(sel, qsm_ref[...], (((1,), (1,)), ((), ())),
                                preferred_element_type=jnp.float32,
                                precision=_HIGH)
        for c in range(_NS // _SCW):
            qsmt_ref[c] = qsm_t[:, c * _SCW:(c + 1) * _SCW]

        ones_row = jnp.ones((1, _DH), jnp.float32)

        def s_step(i, md):
            chunk = qsmt_ref[i]
            for j in range(_SCW):
                col = chunk[:, j:j + 1]
                ad = jnp.abs(kh - col)
                dj = lax.dot_general(ones_row, ad,
                                     (((1,), (0,)), ((), ())),
                                     preferred_element_type=jnp.float32,
                                     precision=_HIGH)
                md = jnp.minimum(md, dj)
            return md

        md = lax.fori_loop(0, _NS // _SCW, s_step,
                           jnp.full((1, _NKV), jnp.inf, jnp.float32))

        ib = lax.bitcast_convert_type(md, jnp.int32)
        one = jnp.int32(1)

        def bit_step(t, v):
            bit = 30 - t
            cand = v | (jnp.left_shift(one, bit) - 1)
            cnt = jnp.sum((ib <= cand).astype(jnp.int32))
            return jnp.where(cnt >= _TOPK, v, v | jnp.left_shift(one, bit))

        vstar = lax.fori_loop(0, 31, bit_step, jnp.int32(0))
        thr = lax.bitcast_convert_type(vstar, jnp.float32)
        keep_ref[0, :] = (md[0] <= thr).astype(jnp.float32)

    keep = keep_ref[0, :] > 0.5
    qh = qh_ref[:, :_DH]
    logits = lax.dot_general(qh, kh, (((1,), (0,)), ((), ())),
                             preferred_element_type=jnp.float32)
    logits = jnp.where(keep[None, :], logits, jnp.float32(-1e30))
    mx = jnp.max(logits, axis=1, keepdims=True)
    p = jnp.exp(logits - mx)
    attn = p / jnp.sum(p, axis=1, keepdims=True)
    att = lax.dot_general(attn, v_ref[...], (((1,), (1,)), ((), ())),
                          preferred_element_type=jnp.float32)
    contrib = lax.dot_general(wo_ref[0], att, (((1,), (1,)), ((), ())),
                              preferred_element_type=jnp.float32)

    @pl.when(h == 0)
    def _():
        acc_ref[:, pl.ds(qb * _QB, _QB)] = contrib

    @pl.when(h > 0)
    def _():
        acc_ref[:, pl.ds(qb * _QB, _QB)] = (
            acc_ref[:, pl.ds(qb * _QB, _QB)] + contrib)

    @pl.when(h == _HEADS - 1)
    def _():
        ln = _chan_ln(acc_ref[:, pl.ds(qb * _QB, _QB)], og_ref[...],
                      ob_ref[...])
        out_ref[...] = gam_ref[0, 0] * ln + qs_ref[...]


def _main_call(k, v, qhm, qsm, W_out, qs2, og, ob, gam, interpret=False):
    W_out = W_out.reshape(_DIM, _HEADS, _DH).transpose(1, 0, 2)
    return pl.pallas_call(
        _main_body,
        grid=(_HEADS, _NQB),
        in_specs=[
            pl.BlockSpec((_DH, _NKV), lambda h, qb: (h, 0)),
            pl.BlockSpec((_DH, _NKV), lambda h, qb: (h, 0)),
            pl.BlockSpec((_QB, _DHP), lambda h, qb: (h * _NQB + qb, 0)),
            pl.BlockSpec((_NS, _DHP), lambda h, qb: (h, 0)),
            pl.BlockSpec((1, _DIM, _DH), lambda h, qb: (h, 0, 0)),
            pl.BlockSpec((_DIM, _QB), lambda h, qb: (0, qb)),
            pl.BlockSpec((_DIM, 1), lambda h, qb: (0, 0)),
            pl.BlockSpec((_DIM, 1), lambda h, qb: (0, 0)),
            pl.BlockSpec((1, 1), lambda h, qb: (0, 0)),
        ],
        out_specs=pl.BlockSpec((_DIM, _QB), lambda h, qb: (0, qb)),
        out_shape=jax.ShapeDtypeStruct((_DIM, _NQ), jnp.float32),
        scratch_shapes=[pltpu.VMEM((_DIM, _NQ), jnp.float32),
                        pltpu.VMEM((1, _NKV), jnp.float32),
                        pltpu.VMEM((_NS // _SCW, _DH, _SCW), jnp.float32)],
        interpret=interpret,
    )(k, v, qhm, qsm, W_out, qs2, og, ob, gam)


def kernel(query_source, context, W_q, W_kv, W_out, ctx_g, ctx_b, qs_g, qs_b,
           out_g, out_b, gamma):
    ctx2 = context.reshape(_DIM, _NKV)
    qs2 = query_source.reshape(_DIM, _NQ)
    col = lambda t: t.reshape(_DIM, 1)
    k, v, qhm = _prep_call(ctx2, qs2, W_kv, W_q, col(ctx_g), col(ctx_b),
                           col(qs_g), col(qs_b))
    qsm = _sc_gather_fn()(qhm, jnp.asarray(_FLAT_IDX))
    out2 = _main_call(k, v, qhm, qsm, W_out, qs2, col(out_g), col(out_b),
                      gamma.reshape(1, 1))
    return out2.reshape(1, _DIM, 32, 32)


# speedup vs baseline: 2.1565x; 2.1565x over previous
import functools

import numpy as np
import jax
import jax.numpy as jnp
from jax import lax
from jax.experimental import pallas as pl
from jax.experimental.pallas import tpu as pltpu
from jax.experimental.pallas import tpu_sc as plsc

_DIM = 384
_HEADS = 8
_DH = 64
_INNER = 512
_TOPK = 512
_NQ = 1024
_NKV = 4096
_DHP = 128
_HIGH = jax.lax.Precision.HIGHEST


_SAMPLE_IDX = np.array([
    [6,8,11,13,17,18,20,23,25,26,28,34,37,41,45,46,48,51,56,58,60,62,66,67,69,70,71,74,80,83,84,87,91,92,93,95,97,99,101,108,112,114,120,125,127,128,131,134,135,136,143,145,146,147,149,153,154,164,168,171,172,173,183,184,185,186,192,193,196,198,199,201,206,207,209,212,219,223,224,225,226,228,232,235,238,239,240,247,248,249,250,252,254,258,261,264,266,267,268,269,276,277,278,279,284,285,288,291,293,295,297,299,300,302,303,304,308,309,311,316,318,321,324,325,332,334,335,337,338,342,345,348,352,353,354,356,357,361,363,365,370,371,372,377,381,383,385,386,388,389,395,397,402,406,408,409,410,411,414,415,420,422,424,428,431,433,436,437,438,439,440,441,446,448,451,453,456,459,460,463,465,467,469,472,474,475,479,484,487,489,492,495,496,497,500,501,502,503,505,508,518,519,523,525,527,528,529,530,535,539,543,544,545,546,548,549,552,556,561,562,564,566,567,568,571,574,576,578,579,580,587,588,591,592,593,595,596,598,601,604,606,608,609,611,612,617,619,620,625,626,629,634,641,643,647,648,650,651,653,654,660,661,663,665,667,674,676,682,683,684,686,691,692,694,695,696,698,700,706,707,714,716,720,729,730,731,733,735,737,740,742,749,750,756,758,759,760,762,764,767,771,772,774,775,776,779,784,786,788,789,790,792,793,795,798,799,800,802,803,804,807,808,812,814,815,816,817,819,820,823,824,827,828,830,832,833,834,835,837,839,840,842,845,849,856,857,859,860,861,862,865,869,871,874,878,889,891,892,894,895,898,902,904,906,913,922,924,928,930,933,938,944,946,948,953,955,961,967,969,971,972,974,976,978,980,985,986,993,999,1000,1003,1005,1008,1010,1011,1013,1015,1016,1018,1019,1021,1022,1023,6,6,6,6,6,6,6,6,6,6,6,6,6],
    [2,5,11,12,14,37,40,41,42,44,46,48,49,53,55,56,59,64,65,68,70,75,77,81,82,86,91,96,97,98,101,104,107,111,114,117,124,125,128,129,130,135,136,138,140,147,150,152,153,154,156,160,161,163,164,167,169,172,175,183,184,186,187,188,189,191,196,197,201,206,208,210,212,213,214,216,218,220,221,223,227,229,230,236,237,240,246,248,249,251,254,256,258,260,264,266,267,270,272,277,278,279,280,281,286,287,290,294,296,298,299,302,304,307,309,317,318,320,325,326,328,330,331,332,335,336,338,339,340,343,345,347,348,355,357,358,364,365,366,367,369,370,371,374,376,378,381,383,387,389,391,399,400,405,407,408,412,414,415,419,421,426,429,430,431,433,436,437,444,445,449,450,453,454,457,458,459,462,463,468,469,475,476,477,478,480,481,485,487,490,491,494,496,498,501,507,510,511,512,515,520,525,528,530,533,534,541,543,544,547,548,554,556,557,558,559,561,568,569,570,574,583,586,592,593,597,599,601,605,606,609,610,611,613,618,620,623,624,625,633,635,637,638,640,641,642,643,645,649,653,654,657,660,661,663,664,669,671,673,674,679,689,698,700,703,704,709,710,711,712,714,718,719,721,722,723,726,727,729,730,732,733,738,740,741,742,745,746,750,754,756,757,758,762,768,770,773,774,775,778,781,784,785,787,789,791,792,793,794,799,802,804,810,813,814,823,824,825,828,831,838,842,843,844,847,849,850,853,858,859,860,862,863,864,865,873,874,875,876,880,883,885,886,887,891,892,894,895,897,899,903,905,907,910,915,923,926,930,934,937,938,940,943,947,949,959,960,962,963,965,966,968,972,974,984,987,988,989,990,991,1001,1003,1005,1006,1008,1010,1011,1012,1016,1017,1018,1019,2,2,2,2,2,2,2,2,2,2,2,2,2,2,2,2,2,2,2,2,2,2,2,2],
    [2,4,6,8,9,10,15,18,21,23,24,25,26,32,34,37,39,43,44,46,47,48,50,58,59,61,63,64,67,71,72,77,82,84,85,89,93,94,95,97,98,101,103,106,112,113,120,122,124,127,129,131,132,133,134,135,137,138,151,155,156,159,162,163,165,166,171,172,173,178,182,186,188,189,197,198,203,206,208,211,215,217,220,221,224,226,229,233,237,245,250,254,259,260,264,265,267,268,270,271,272,274,276,279,280,281,284,290,293,294,296,297,299,302,304,305,308,312,317,322,324,325,326,327,330,333,336,340,342,345,346,347,349,352,353,356,357,358,360,367,368,369,372,378,379,382,383,385,386,387,390,392,394,395,396,397,398,399,402,403,404,405,407,408,410,411,412,414,417,418,419,421,422,428,429,434,435,437,438,444,452,455,457,458,461,462,465,469,470,471,475,479,480,481,483,486,487,488,492,498,501,503,505,511,520,521,524,527,541,542,544,548,550,551,555,556,557,560,564,567,569,573,575,579,580,581,584,585,587,589,590,591,593,595,596,597,599,607,613,617,621,623,624,625,626,631,638,639,640,642,646,647,649,650,652,656,657,660,661,667,669,670,671,673,680,681,683,684,686,690,691,692,693,694,695,696,697,701,707,708,709,710,713,714,715,717,719,720,721,722,729,731,734,735,736,743,744,750,751,754,757,759,760,762,763,764,765,766,768,774,775,777,782,785,786,787,788,789,792,794,796,797,799,802,803,811,812,813,815,818,820,821,823,824,825,826,828,829,831,833,836,837,838,839,840,841,845,846,848,850,852,853,854,856,859,860,864,867,869,870,874,877,878,879,882,883,884,890,891,892,893,903,905,908,911,914,919,920,925,926,927,928,930,931,936,938,939,940,946,948,949,950,953,956,959,962,964,969,970,973,974,985,988,990,993,994,996,997,999,1002,1004,1007,1010,1016,2,2],
    [0,2,4,10,13,17,18,19,20,28,30,33,34,37,38,42,43,44,45,48,51,53,55,56,57,61,62,64,65,68,79,81,82,83,87,92,98,99,103,104,106,111,112,114,117,124,126,127,133,134,135,141,142,144,148,149,151,155,156,157,158,161,162,163,166,167,171,173,178,179,184,186,188,189,190,191,199,206,209,210,213,217,221,224,225,226,229,230,233,235,236,237,238,240,245,249,252,256,261,262,263,264,265,267,270,273,274,278,280,281,282,285,288,289,291,293,297,299,301,302,303,308,310,311,313,314,321,322,326,327,329,330,335,336,344,347,349,352,358,363,367,369,380,383,386,387,389,397,402,404,406,407,408,410,411,413,414,415,423,424,427,429,430,436,440,443,444,446,447,451,453,456,459,463,465,469,473,474,475,476,480,481,482,483,485,486,488,489,491,495,497,498,500,501,503,505,506,507,508,509,510,519,522,523,526,527,529,530,531,532,533,534,536,537,538,541,543,544,546,552,553,554,557,559,560,561,563,566,569,571,572,575,576,583,586,587,588,592,593,595,596,598,601,602,603,604,606,610,611,613,614,617,621,622,623,624,627,628,637,639,642,643,645,648,652,656,662,663,671,674,675,680,689,691,694,695,699,700,701,703,705,707,710,712,713,716,718,719,721,722,723,726,728,732,733,734,735,744,745,746,749,751,753,760,767,768,770,772,775,780,783,785,786,789,799,800,801,804,806,814,818,822,823,827,828,830,832,837,838,839,841,842,843,849,856,859,861,862,863,865,870,876,880,888,889,891,895,896,897,899,900,906,907,908,910,911,912,915,916,918,919,921,922,929,930,933,934,936,938,939,941,943,944,946,950,952,955,958,960,963,964,966,969,971,973,974,975,976,977,978,979,980,981,985,986,988,990,994,998,999,1000,1001,1002,1008,1010,1014,1015,1017,1021,1022,0,0,0,0,0,0],
    [0,1,2,3,5,7,8,12,18,22,23,24,25,26,30,33,41,43,44,47,50,56,57,59,60,61,62,65,66,67,69,71,73,74,78,79,80,81,82,87,90,92,94,96,98,100,102,103,106,107,108,113,117,121,122,125,126,127,128,130,133,136,143,145,148,149,150,152,154,156,161,165,167,168,170,172,179,181,184,185,186,188,196,204,207,215,217,218,229,230,233,234,235,236,237,242,244,245,246,247,252,254,256,258,261,262,264,266,267,269,272,273,274,275,277,285,287,295,296,297,299,300,302,303,306,310,311,312,313,314,315,319,321,330,334,335,337,338,340,355,358,359,361,362,365,367,368,369,371,374,375,376,380,382,383,384,389,393,395,400,401,403,405,407,408,410,413,415,417,419,423,424,425,426,428,430,432,435,438,440,442,443,450,452,455,459,462,464,465,471,472,474,475,476,477,478,480,483,488,489,492,493,495,496,498,499,500,503,505,512,513,514,518,520,524,531,532,538,539,541,544,551,552,556,557,558,560,563,564,567,569,582,586,588,591,592,596,602,605,606,607,614,615,621,625,627,631,633,638,639,643,644,646,647,650,651,654,656,659,662,665,672,674,678,682,686,687,689,692,696,697,698,703,705,707,711,715,718,721,723,726,728,732,736,739,741,743,758,763,765,770,772,774,778,785,786,787,790,794,799,800,806,812,813,816,823,824,825,829,832,833,841,844,846,847,851,860,861,862,863,864,865,866,869,870,874,876,882,884,885,886,887,891,894,895,896,898,900,901,903,904,906,907,908,912,913,914,916,917,920,924,926,928,930,931,932,934,937,939,940,941,942,946,950,951,953,956,958,960,962,963,964,966,969,973,977,978,979,980,981,984,988,994,1000,1001,1005,1007,1010,1011,1013,1016,1020,1021,1023,0,0,0,0,0,0,0,0,0,0,0,0,0,0,0,0,0,0,0,0,0,0],
    [8,9,11,14,15,17,18,19,20,22,35,36,38,39,40,42,43,45,49,50,55,56,61,62,63,68,71,75,78,79,84,87,97,99,100,102,103,105,106,114,124,127,130,132,136,137,138,140,144,145,146,148,151,155,156,157,162,171,174,178,180,183,184,186,187,190,196,197,198,204,205,209,210,211,213,214,221,222,225,228,229,233,235,239,241,242,243,244,247,248,250,252,253,254,255,256,257,260,263,266,268,270,271,273,274,276,277,278,281,284,285,286,288,289,291,297,298,302,307,309,312,313,317,318,320,323,324,327,328,332,335,346,347,359,364,365,369,371,374,375,377,380,382,384,386,387,388,389,391,392,395,400,404,405,406,409,413,415,417,418,420,422,424,425,426,428,430,431,432,434,437,438,441,442,445,446,451,452,453,455,456,464,466,468,470,471,474,475,476,477,478,479,482,483,492,495,497,498,501,502,503,510,511,515,516,519,523,524,525,527,530,531,532,535,538,539,541,542,543,546,557,559,565,567,570,573,578,581,584,587,592,596,597,598,601,602,604,605,607,611,614,618,620,626,627,628,630,631,638,641,643,644,647,651,652,658,659,662,665,668,673,674,676,677,678,679,682,684,685,694,700,701,704,706,711,712,714,715,717,721,724,725,726,732,734,735,736,738,742,743,744,746,749,750,752,757,761,763,765,766,771,773,775,777,779,780,781,787,788,789,794,795,798,801,803,805,809,812,815,816,821,822,825,826,827,828,833,834,836,838,840,842,844,847,852,854,855,857,858,868,869,870,871,873,874,875,877,880,881,883,885,888,890,896,897,898,899,903,904,910,912,914,916,917,919,923,924,926,927,933,936,938,943,944,950,959,960,962,963,968,974,975,977,983,984,985,986,987,989,990,992,994,998,999,1001,1002,1003,1006,1007,1010,1011,1013,1022,8,8,8,8,8,8,8,8,8,8,8,8,8],
    [0,1,3,4,5,8,9,11,12,15,16,18,26,30,32,33,42,44,48,53,54,56,58,59,60,63,68,70,72,78,80,82,83,85,90,95,96,97,101,103,105,109,111,121,122,134,136,139,140,141,144,150,152,157,158,159,160,163,164,169,171,172,173,174,175,178,181,182,183,184,186,189,192,193,195,197,198,199,200,206,209,211,213,216,218,219,220,222,225,227,231,233,234,235,236,238,241,245,247,248,249,250,255,256,258,261,262,265,269,270,276,277,284,286,292,293,295,296,300,306,307,308,309,311,313,320,328,332,333,334,339,340,342,345,346,349,352,355,357,359,360,362,368,370,375,376,378,382,385,386,387,391,397,407,411,412,416,417,425,430,431,436,438,439,440,441,443,444,450,453,454,456,457,459,461,462,464,466,467,469,470,471,473,476,478,479,480,481,482,483,490,491,493,494,497,498,499,500,501,505,508,509,511,514,515,519,522,524,529,530,534,537,550,551,555,557,558,559,560,561,568,570,573,580,589,593,595,597,598,599,602,603,604,610,623,624,625,627,629,634,636,640,641,645,647,649,656,657,658,659,660,661,662,666,667,668,675,677,680,683,684,685,686,689,693,697,700,703,711,715,716,721,724,725,728,729,732,737,738,739,740,745,746,747,748,750,752,754,755,756,757,759,761,764,772,779,780,784,786,791,792,794,795,802,805,806,807,809,810,812,813,815,818,819,822,824,825,833,834,836,838,839,840,842,843,846,847,848,849,852,857,859,860,868,871,872,877,879,880,881,882,883,885,886,893,895,898,900,901,903,906,907,911,913,914,916,917,925,926,928,929,932,933,938,940,947,950,954,955,958,959,963,964,967,968,970,974,975,976,977,979,981,984,987,991,992,993,994,997,998,999,1000,1001,1003,1005,1008,1009,1010,1014,1017,1018,0,0,0,0,0,0,0,0,0,0,0,0,0,0,0],
    [0,6,14,17,20,21,26,27,28,30,35,38,40,41,42,45,52,57,59,61,62,63,66,71,73,76,77,79,80,83,86,87,88,89,90,91,96,100,103,105,108,110,117,118,122,123,127,130,131,132,134,135,136,139,140,143,146,149,150,154,158,159,160,175,177,181,182,189,191,193,198,199,202,204,206,207,209,210,212,213,216,217,219,221,225,227,230,234,240,243,244,246,247,250,254,264,265,266,268,269,270,275,278,281,283,284,287,288,289,293,294,295,301,302,304,305,309,312,320,322,327,331,332,336,338,343,347,350,355,359,360,361,363,370,371,372,373,376,377,379,385,391,392,394,395,397,399,401,406,417,421,425,429,432,438,439,440,441,443,445,446,449,456,457,458,462,470,472,476,477,484,486,488,489,494,499,501,504,505,508,522,526,530,533,536,537,538,539,540,544,545,549,550,552,553,554,557,562,564,567,568,569,573,574,578,579,580,581,583,585,589,590,591,593,594,595,597,599,600,602,605,607,608,612,614,616,617,625,626,627,636,638,639,640,643,652,653,655,656,657,660,661,664,667,670,678,680,681,684,687,689,691,693,694,695,696,701,703,704,711,712,715,718,720,721,722,723,725,727,729,730,731,732,734,736,739,740,743,747,749,750,751,752,755,757,759,760,761,763,765,767,770,771,773,774,776,777,789,790,794,795,797,799,800,801,805,806,807,810,812,814,817,818,819,822,825,829,830,835,842,844,850,854,858,859,860,861,862,863,865,867,868,870,873,875,876,879,880,881,883,885,893,894,896,897,898,899,901,906,909,910,911,915,917,918,921,923,924,929,930,933,935,938,946,947,948,950,952,956,957,958,961,965,968,969,971,973,977,979,982,985,990,993,1000,1001,1002,1004,1011,1012,1013,1016,1017,1020,1021,1022,0,0,0,0,0,0,0,0,0,0,0,0,0,0,0,0,0,0,0,0,0],
], dtype=np.int32)

_NS = _SAMPLE_IDX.shape[1]
_FLAT_IDX = (_SAMPLE_IDX + (np.arange(_HEADS, dtype=np.int32) * _NQ)[:, None]
             ).reshape(-1).astype(np.int32)
_NSC = 32
_PER_W = _FLAT_IDX.shape[0] // _NSC


def _chan_ln(x, g, b):
    m = jnp.mean(x, axis=0, keepdims=True)
    d = x - m
    var = jnp.mean(d * d, axis=0, keepdims=True)
    return d / jnp.sqrt(var + 1e-5) * g + b


def _prep_body(ctx_ref, qs_ref, wkv_ref, wq_ref, cg_ref, cb_ref, qg_ref,
               qb_ref, k_ref, v_ref, qhm_ref):
    ctxn = _chan_ln(ctx_ref[...], cg_ref[...], cb_ref[...])
    qsn = _chan_ln(qs_ref[...], qg_ref[...], qb_ref[...])

    kv = lax.dot_general(wkv_ref[...], ctxn, (((1,), (0,)), ((), ())),
                         preferred_element_type=jnp.float32)
    q = lax.dot_general(wq_ref[...], qsn, (((1,), (0,)), ((), ())),
                        preferred_element_type=jnp.float32)

    def l2n(x, n_pos):
        xr = x.reshape(_HEADS, _DH, n_pos)
        n = jnp.sqrt(jnp.sum(xr * xr, axis=1, keepdims=True))
        return (xr / jnp.maximum(n, 1e-12)).reshape(_HEADS * _DH, n_pos)

    k_ref[...] = l2n(kv[:_INNER], _NKV)
    v_ref[...] = kv[_INNER:]
    qn = l2n(q, _NQ)

    rows = lax.broadcasted_iota(jnp.int32, (_DH, _DHP), 0)
    cols = lax.broadcasted_iota(jnp.int32, (_DH, _DHP), 1)
    eye = (rows == cols).astype(jnp.float32)
    for h in range(_HEADS):
        qh = qn[h * _DH:(h + 1) * _DH]
        qht = lax.dot_general(qh, eye, (((0,), (0,)), ((), ())),
                              preferred_element_type=jnp.float32,
                              precision=_HIGH)
        qhm_ref[h * _NQ:(h + 1) * _NQ, :] = qht


def _prep_call(ctx2, qs2, W_kv, W_q, cg, cb, qg, qb, interpret=False):
    return pl.pallas_call(
        _prep_body,
        out_shape=[
            jax.ShapeDtypeStruct((_INNER, _NKV), jnp.float32),
            jax.ShapeDtypeStruct((_INNER, _NKV), jnp.float32),
            jax.ShapeDtypeStruct((_HEADS * _NQ, _DHP), jnp.float32),
        ],
        interpret=interpret,
    )(ctx2, qs2, W_kv, W_q, cg, cb, qg, qb)


@functools.cache
def _sc_gather_fn():
    @functools.partial(
        pl.kernel,
        out_type=jax.ShapeDtypeStruct((_HEADS * _NS, _DHP), jnp.float32),
        mesh=plsc.VectorSubcoreMesh(core_axis_name="c", subcore_axis_name="s"),
        scratch_types=[
            pltpu.VMEM((_PER_W,), jnp.int32),
            pltpu.VMEM((_PER_W, _DHP), jnp.float32),
            pltpu.SemaphoreType.DMA,
        ],
    )
    def _sc_gather(qhm_hbm, idx_hbm, out_hbm, idx_v, rows_v, sem):
        wid = lax.axis_index("s") * 2 + lax.axis_index("c")
        base = wid * _PER_W
        pltpu.sync_copy(idx_hbm.at[pl.ds(base, _PER_W)], idx_v)
        pltpu.async_copy(qhm_hbm.at[idx_v], rows_v, sem).wait()
        pltpu.sync_copy(rows_v, out_hbm.at[pl.ds(base, _PER_W)])

    return _sc_gather


_NQB = 4
_QB = _NQ // _NQB
_SCW = 16


def _main_body(k_ref, v_ref, qh_ref, qsm_ref, wo_ref, qs_ref, og_ref, ob_ref,
               gam_ref, out_ref, acc_ref, keep_ref, qsmt_ref):
    h = pl.program_id(0)
    qb = pl.program_id(1)
    kh = k_ref[...]

    @pl.when(qb == 0)
    def _():
        rows = lax.broadcasted_iota(jnp.int32, (_DH, _DHP), 0)
        cols = lax.broadcasted_iota(jnp.int32, (_DH, _DHP), 1)
        sel = (rows == cols).astype(jnp.float32)
        qsm_t = lax.dot_general(sel, qsm_ref[...], (((1,), (1,)), ((), ())),
                                preferred_element_type=jnp.float32,
                                precision=_HIGH)
        for c in range(_NS // _SCW):
            qsmt_ref[c] = qsm_t[:, c * _SCW:(c + 1) * _SCW]

        def s_step(i, md):
            chunk = qsmt_ref[i]
            for j in range(_SCW):
                col = chunk[:, j:j + 1]
                d = jnp.sum(jnp.abs(kh - col), axis=0,
                            keepdims=True)
                md = jnp.minimum(md, d)
            return md

        md = lax.fori_loop(0, _NS // _SCW, s_step,
                           jnp.full((1, _NKV), jnp.inf, jnp.float32))

        ib = lax.bitcast_convert_type(md, jnp.int32)
        one = jnp.int32(1)

        def bit_step(t, v):
            bit = 30 - t
            cand = v | (jnp.left_shift(one, bit) - 1)
            cnt = jnp.sum((ib <= cand).astype(jnp.int32))
            return jnp.where(cnt >= _TOPK, v, v | jnp.left_shift(one, bit))

        vstar = lax.fori_loop(0, 31, bit_step, jnp.int32(0))
        thr = lax.bitcast_convert_type(vstar, jnp.float32)
        keep_ref[0, :] = (md[0] <= thr).astype(jnp.float32)

    keep = keep_ref[0, :] > 0.5
    qh = qh_ref[:, :_DH]
    logits = lax.dot_general(qh, kh, (((1,), (0,)), ((), ())),
                             preferred_element_type=jnp.float32)
    logits = jnp.where(keep[None, :], logits, jnp.float32(-1e30))
    mx = jnp.max(logits, axis=1, keepdims=True)
    p = jnp.exp(logits - mx)
    attn = p / jnp.sum(p, axis=1, keepdims=True)
    att = lax.dot_general(attn, v_ref[...], (((1,), (1,)), ((), ())),
                          preferred_element_type=jnp.float32)
    contrib = lax.dot_general(wo_ref[0], att, (((1,), (1,)), ((), ())),
                              preferred_element_type=jnp.float32)

    @pl.when(h == 0)
    def _():
        acc_ref[:, pl.ds(qb * _QB, _QB)] = contrib

    @pl.when(h > 0)
    def _():
        acc_ref[:, pl.ds(qb * _QB, _QB)] = (
            acc_ref[:, pl.ds(qb * _QB, _QB)] + contrib)

    @pl.when(h == _HEADS - 1)
    def _():
        ln = _chan_ln(acc_ref[:, pl.ds(qb * _QB, _QB)], og_ref[...],
                      ob_ref[...])
        out_ref[...] = gam_ref[0, 0] * ln + qs_ref[...]


def _main_call(k, v, qhm, qsm, W_out, qs2, og, ob, gam, interpret=False):
    W_out = W_out.reshape(_DIM, _HEADS, _DH).transpose(1, 0, 2)
    return pl.pallas_call(
        _main_body,
        grid=(_HEADS, _NQB),
        in_specs=[
            pl.BlockSpec((_DH, _NKV), lambda h, qb: (h, 0)),
            pl.BlockSpec((_DH, _NKV), lambda h, qb: (h, 0)),
            pl.BlockSpec((_QB, _DHP), lambda h, qb: (h * _NQB + qb, 0)),
            pl.BlockSpec((_NS, _DHP), lambda h, qb: (h, 0)),
            pl.BlockSpec((1, _DIM, _DH), lambda h, qb: (h, 0, 0)),
            pl.BlockSpec((_DIM, _QB), lambda h, qb: (0, qb)),
            pl.BlockSpec((_DIM, 1), lambda h, qb: (0, 0)),
            pl.BlockSpec((_DIM, 1), lambda h, qb: (0, 0)),
            pl.BlockSpec((1, 1), lambda h, qb: (0, 0)),
        ],
        out_specs=pl.BlockSpec((_DIM, _QB), lambda h, qb: (0, qb)),
        out_shape=jax.ShapeDtypeStruct((_DIM, _NQ), jnp.float32),
        scratch_shapes=[pltpu.VMEM((_DIM, _NQ), jnp.float32),
                        pltpu.VMEM((1, _NKV), jnp.float32),
                        pltpu.VMEM((_NS // _SCW, _DH, _SCW), jnp.float32)],
        interpret=interpret,
    )(k, v, qhm, qsm, W_out, qs2, og, ob, gam)


def kernel(query_source, context, W_q, W_kv, W_out, ctx_g, ctx_b, qs_g, qs_b,
           out_g, out_b, gamma):
    ctx2 = context.reshape(_DIM, _NKV)
    qs2 = query_source.reshape(_DIM, _NQ)
    col = lambda t: t.reshape(_DIM, 1)
    k, v, qhm = _prep_call(ctx2, qs2, W_kv, W_q, col(ctx_g), col(ctx_b),
                           col(qs_g), col(qs_b))
    qsm = _sc_gather_fn()(qhm, jnp.asarray(_FLAT_IDX))
    out2 = _main_call(k, v, qhm, qsm, W_out, qs2, col(out_g), col(out_b),
                      gamma.reshape(1, 1))
    return out2.reshape(1, _DIM, 32, 32)
